# baseline (speedup 1.0000x reference)
import jax
import jax.numpy as jnp
import numpy as np
from jax.experimental import pallas as pl
from jax.experimental.pallas import tpu as pltpu
from jax.sharding import Mesh, PartitionSpec as P

_D = 512
_K = 1024
_Q = 8
_TILE = 512


_H = _TILE // 2


def _rvq_body(x_ref, c1_ref, ccat_ref, cnorm_ref,
              qout_ref, idx_ref, loss_ref):
    iota = jax.lax.broadcasted_iota(jnp.int32, (_H, _K), 1)
    residual = [x_ref[0, :_H], x_ref[0, _H:]]
    quant_acc = [jnp.zeros_like(residual[0]) for _ in range(2)]
    fnorm = [jnp.sum(r * r, axis=1, keepdims=True) for r in residual]
    idx_cols = [[], []]
    loss_rows = []
    for q in range(_Q):
        mm = [jax.lax.dot_general(
            residual[h].astype(jnp.bfloat16), c1_ref[q],
            (((1,), (1,)), ((), ())),
            preferred_element_type=jnp.float32) for h in range(2)]
        loss_q = []
        for h in range(2):
            d2 = (fnorm[h] - 2.0 * mm[h]) + cnorm_ref[q]
            idxk = jnp.argmin(d2, axis=1, keepdims=True)
            onehot = (iota == idxk).astype(jnp.bfloat16)
            qcat = jax.lax.dot_general(onehot, ccat_ref[q],
                                       (((1,), (0,)), ((), ())),
                                       preferred_element_type=jnp.float32)
            quant = (qcat[:, :_D] + qcat[:, _D:2 * _D]) + qcat[:, 2 * _D:]
            rmq = residual[h] - quant
            rn = jnp.sum(rmq * rmq, axis=1, keepdims=True)
            loss_q.append(jnp.sum(rn, keepdims=True))
            quant_acc[h] = quant_acc[h] + (residual[h] + (quant - residual[h]))
            residual[h] = rmq
            fnorm[h] = rn
            idx_cols[h].append(idxk)
        loss_rows.append(loss_q[0] + loss_q[1])
    qout_ref[0] = jnp.concatenate([quant_acc[0], quant_acc[1]], axis=0)
    idx_ref[0] = jnp.concatenate(
        [jnp.concatenate(idx_cols[h], axis=1) for h in range(2)], axis=0)
    loss_ref[0] = jnp.concatenate(loss_rows, axis=0)


def kernel(clap_embeddings, codebooks):
    x = clap_embeddings
    B, T, D = x.shape
    n_tiles = (B * T) // _TILE

    c1 = codebooks.astype(jnp.bfloat16)
    r1 = codebooks - jax.lax.optimization_barrier(c1).astype(jnp.float32)
    c2 = r1.astype(jnp.bfloat16)
    r2 = r1 - jax.lax.optimization_barrier(c2).astype(jnp.float32)
    c3 = r2.astype(jnp.bfloat16)
    ccat = jnp.concatenate([c1, c2, c3], axis=2)
    cnorm = jnp.sum(codebooks * codebooks, axis=2)[:, None, :]

    xt = x.reshape(n_tiles, _TILE, D)

    const3 = lambda i: (0, 0, 0)

    def _call(xt_l, c1_l, ccat_l, cnorm_l):
        nt = xt_l.shape[0]
        return pl.pallas_call(
            _rvq_body,
            grid=(nt,),
            in_specs=[
                pl.BlockSpec((1, _TILE, _D), lambda i: (i, 0, 0)),
                pl.BlockSpec((_Q, _K, _D), const3),
                pl.BlockSpec((_Q, _K, 3 * _D), const3),
                pl.BlockSpec((_Q, 1, _K), const3),
            ],
            out_specs=[
                pl.BlockSpec((1, _TILE, _D), lambda i: (i, 0, 0)),
                pl.BlockSpec((1, _TILE, _Q), lambda i: (i, 0, 0)),
                pl.BlockSpec((1, _Q, 1), lambda i: (i, 0, 0)),
            ],
            out_shape=[
                jax.ShapeDtypeStruct((nt, _TILE, _D), jnp.float32),
                jax.ShapeDtypeStruct((nt, _TILE, _Q), jnp.int32),
                jax.ShapeDtypeStruct((nt, _Q, 1), jnp.float32),
            ],
            compiler_params=pltpu.CompilerParams(
                dimension_semantics=("arbitrary",),
                vmem_limit_bytes=60000 * 1024,
            ),
        )(xt_l, c1_l, ccat_l, cnorm_l)

    devs = jax.devices()
    ndev = 2 if len(devs) >= 2 and n_tiles % 2 == 0 else 1
    mesh = Mesh(np.array(devs[:ndev]), ("d",))
    qout, idx, loss = jax.shard_map(
        _call, mesh=mesh, check_vma=False,
        in_specs=(P("d", None, None), P(None, None, None),
                  P(None, None, None), P(None, None, None)),
        out_specs=(P("d", None, None), P("d", None, None),
                   P("d", None, None)),
    )(xt, c1, ccat, cnorm)

    quantized_out = qout.reshape(B, T, D)
    indices = idx.reshape(B, T, _Q)
    commit_loss = jnp.sum(loss) / jnp.float32(B * T * D)
    return quantized_out, indices, commit_loss


# speedup vs baseline: 1.7522x; 1.7522x over previous
import jax
import jax.numpy as jnp
import numpy as np
from jax.experimental import pallas as pl
from jax.experimental.pallas import tpu as pltpu
from jax.sharding import Mesh, PartitionSpec as P

_D = 512
_K = 1024
_Q = 8
_TILE = 1024


_H = _TILE // 2


def _rvq_body(x_ref, c1_ref, ccat_ref, cnorm_ref,
              qout_ref, idx_ref, loss_ref):
    iota = jax.lax.broadcasted_iota(jnp.int32, (_H, _K), 1)
    residual = [x_ref[0, :_H], x_ref[0, _H:]]
    quant_acc = [jnp.zeros_like(residual[0]) for _ in range(2)]
    fnorm = [jnp.sum(r * r, axis=1, keepdims=True) for r in residual]
    idx_cols = [[], []]
    loss_rows = []
    for q in range(_Q):
        mm = [jax.lax.dot_general(
            residual[h].astype(jnp.bfloat16), c1_ref[q],
            (((1,), (1,)), ((), ())),
            preferred_element_type=jnp.float32) for h in range(2)]
        loss_q = []
        for h in range(2):
            d2 = (fnorm[h] - 2.0 * mm[h]) + cnorm_ref[q]
            idxk = jnp.argmin(d2, axis=1, keepdims=True)
            onehot = (iota == idxk).astype(jnp.bfloat16)
            qcat = jax.lax.dot_general(onehot, ccat_ref[q],
                                       (((1,), (0,)), ((), ())),
                                       preferred_element_type=jnp.float32)
            quant = (qcat[:, :_D] + qcat[:, _D:2 * _D]) + qcat[:, 2 * _D:]
            rmq = residual[h] - quant
            rn = jnp.sum(rmq * rmq, axis=1, keepdims=True)
            loss_q.append(jnp.sum(rn, keepdims=True))
            quant_acc[h] = quant_acc[h] + (residual[h] + (quant - residual[h]))
            residual[h] = rmq
            fnorm[h] = rn
            idx_cols[h].append(idxk)
        loss_rows.append(loss_q[0] + loss_q[1])
    qout_ref[0] = jnp.concatenate([quant_acc[0], quant_acc[1]], axis=0)
    idx_ref[0] = jnp.concatenate(
        [jnp.concatenate(idx_cols[h], axis=1) for h in range(2)], axis=0)
    loss_ref[0] = jnp.concatenate(loss_rows, axis=0)


def kernel(clap_embeddings, codebooks):
    x = clap_embeddings
    B, T, D = x.shape
    n_tiles = (B * T) // _TILE

    c1 = codebooks.astype(jnp.bfloat16)
    r1 = codebooks - jax.lax.optimization_barrier(c1).astype(jnp.float32)
    c2 = r1.astype(jnp.bfloat16)
    r2 = r1 - jax.lax.optimization_barrier(c2).astype(jnp.float32)
    c3 = r2.astype(jnp.bfloat16)
    ccat = jnp.concatenate([c1, c2, c3], axis=2)
    cnorm = jnp.sum(codebooks * codebooks, axis=2)[:, None, :]

    xt = x.reshape(n_tiles, _TILE, D)

    const3 = lambda i: (0, 0, 0)

    def _call(xt_l, c1_l, ccat_l, cnorm_l):
        nt = xt_l.shape[0]
        return pl.pallas_call(
            _rvq_body,
            grid=(nt,),
            in_specs=[
                pl.BlockSpec((1, _TILE, _D), lambda i: (i, 0, 0)),
                pl.BlockSpec((_Q, _K, _D), const3),
                pl.BlockSpec((_Q, _K, 3 * _D), const3),
                pl.BlockSpec((_Q, 1, _K), const3),
            ],
            out_specs=[
                pl.BlockSpec((1, _TILE, _D), lambda i: (i, 0, 0)),
                pl.BlockSpec((1, _TILE, _Q), lambda i: (i, 0, 0)),
                pl.BlockSpec((1, _Q, 1), lambda i: (i, 0, 0)),
            ],
            out_shape=[
                jax.ShapeDtypeStruct((nt, _TILE, _D), jnp.float32),
                jax.ShapeDtypeStruct((nt, _TILE, _Q), jnp.int32),
                jax.ShapeDtypeStruct((nt, _Q, 1), jnp.float32),
            ],
            compiler_params=pltpu.CompilerParams(
                dimension_semantics=("arbitrary",),
                vmem_limit_bytes=60000 * 1024,
            ),
        )(xt_l, c1_l, ccat_l, cnorm_l)

    qout, idx, loss = _call(xt, c1, ccat, cnorm)

    quantized_out = qout.reshape(B, T, D)
    indices = idx.reshape(B, T, _Q)
    commit_loss = jnp.sum(loss) / jnp.float32(B * T * D)
    return quantized_out, indices, commit_loss
